# baseline (speedup 1.0000x reference)
import functools

import jax
import jax.numpy as jnp
import numpy as np
from jax import lax
from jax.experimental import pallas as pl
from jax.experimental.pallas import tpu as pltpu
from jax.experimental.pallas import tpu_sc as plsc

B = 1
S = 2048
T = 2048
H = 768
E = 16
F = 1024
TT = 128
NT = 32
PAD = NT * TT
NW = 32


def _gelu_exact(v):
    a = jnp.abs(v) * np.float32(0.7071067811865476)
    t = 1.0 / (1.0 + np.float32(0.3275911) * a)
    poly = t * (np.float32(0.254829592) + t * (np.float32(-0.284496736)
            + t * (np.float32(1.421413741) + t * (np.float32(-1.453152027)
            + t * np.float32(1.061405429)))))
    erf_a = 1.0 - poly * jnp.exp(-a * a)
    erf_v = jnp.where(v >= 0, erf_a, -erf_a)
    return np.float32(0.5) * v * (1.0 + erf_v)


def _router_body(x_ref, wg_ref, logits_ref, dest_ref, w_ref, te_ref, xc_ref):
    x = x_ref[...]
    xc_ref[...] = x
    wg = wg_ref[...]
    logits = jnp.dot(x, wg, preferred_element_type=jnp.float32)
    logits_ref[...] = logits
    m = jnp.max(logits, axis=1, keepdims=True)
    denom = jnp.sum(jnp.exp(logits - m), axis=1, keepdims=True)
    w_ref[...] = 1.0 / denom
    col = jax.lax.broadcasted_iota(jnp.int32, (T, E), 1)
    eid = jnp.min(jnp.where(logits == m, col, E), axis=1, keepdims=True)
    onehot = (col == eid).astype(jnp.float32)
    r = jax.lax.broadcasted_iota(jnp.int32, (TT, TT), 0)
    c = jax.lax.broadcasted_iota(jnp.int32, (TT, TT), 1)
    tril = (c <= r).astype(jnp.float32)
    blocks = []
    running = jnp.zeros((1, E), jnp.float32)
    for i in range(T // TT):
        csb = jnp.dot(tril, onehot[i * TT:(i + 1) * TT, :],
                      preferred_element_type=jnp.float32) + running
        running = csb[TT - 1:TT, :]
        blocks.append(csb)
    cs = jnp.concatenate(blocks, axis=0)
    counts = running
    padded = jnp.ceil(counts / TT) * TT
    er = jax.lax.broadcasted_iota(jnp.int32, (E, E), 0)
    ec = jax.lax.broadcasted_iota(jnp.int32, (E, E), 1)
    strict = (er < ec).astype(jnp.float32)
    base = jnp.dot(padded, strict, preferred_element_type=jnp.float32)
    pos = jnp.sum(onehot * cs, axis=1, keepdims=True) - 1.0
    tok_base = jnp.sum(onehot * base, axis=1, keepdims=True)
    dest_ref[...] = (tok_base + pos).astype(jnp.int32)
    ti = jax.lax.broadcasted_iota(jnp.int32, (NT, E), 0).astype(jnp.float32) * np.float32(TT)
    ecol = jax.lax.broadcasted_iota(jnp.int32, (NT, E), 1)
    active = jnp.logical_and(ti >= base, ti < base + padded)
    te_ref[...] = jnp.sum(jnp.where(active, ecol, 0), axis=1, keepdims=True)


def _router(x2d, w_gate):
    return pl.pallas_call(
        _router_body,
        out_shape=(
            jax.ShapeDtypeStruct((T, E), jnp.float32),
            jax.ShapeDtypeStruct((T, 1), jnp.int32),
            jax.ShapeDtypeStruct((T, 1), jnp.float32),
            jax.ShapeDtypeStruct((NT, 1), jnp.int32),
            jax.ShapeDtypeStruct((T, H), jnp.float32),
        ),
    )(x2d, w_gate)


@functools.cache
def _sc_kernels():
    mesh = plsc.VectorSubcoreMesh(core_axis_name="c", subcore_axis_name="s")

    @functools.partial(
        pl.kernel,
        out_type=(
            jax.ShapeDtypeStruct((PAD,), jnp.int32),
            jax.ShapeDtypeStruct((PAD,), jnp.float32),
        ),
        mesh=mesh,
        scratch_types=[
            pltpu.VMEM((T,), jnp.int32),
            pltpu.VMEM((T,), jnp.float32),
            pltpu.VMEM((PAD,), jnp.int32),
            pltpu.VMEM((PAD,), jnp.float32),
        ],
        compiler_params=pltpu.CompilerParams(needs_layout_passes=False),
        name="sc_meta",
    )
    def sc_meta(dest_hbm, w_hbm, src_hbm, ws_hbm, dest_v, w_v, src_v, ws_v):
        wid = lax.axis_index("s") * 2 + lax.axis_index("c")

        @pl.when(wid == 0)
        def _():
            pltpu.sync_copy(dest_hbm, dest_v)
            pltpu.sync_copy(w_hbm, w_v)
            lane = lax.broadcasted_iota(jnp.int32, (16,), 0)

            @plsc.parallel_loop(0, T // 16, unroll=8)
            def _scat(i):
                idx = dest_v[pl.ds(i * 16, 16)]
                plsc.store_scatter(src_v, [idx], lane + i * 16)
                plsc.store_scatter(ws_v, [idx], w_v[pl.ds(i * 16, 16)])

            pltpu.sync_copy(src_v, src_hbm)
            pltpu.sync_copy(ws_v, ws_hbm)

    def make_row_gather(n_rows, n_table, clamp, name):
        rpt = n_rows // NW
        ch = 64
        nch = rpt // ch

        @functools.partial(
            pl.kernel,
            out_type=jax.ShapeDtypeStruct((n_rows, H), jnp.float32),
            mesh=mesh,
            scratch_types=[
                pltpu.VMEM((rpt,), jnp.int32),
                pltpu.VMEM((rpt, H), jnp.float32),
                [pltpu.SemaphoreType.DMA] * nch,
            ],
            name=name,
        )
        def k(table_hbm, idx_hbm, out_hbm, idx_v, rows_v, sems):
            wid = lax.axis_index("s") * 2 + lax.axis_index("c")
            base = wid * rpt
            pltpu.sync_copy(idx_hbm.at[pl.ds(base, rpt)], idx_v)
            if clamp:
                for j in range(rpt // 16):
                    v = idx_v[pl.ds(j * 16, 16)]
                    idx_v[pl.ds(j * 16, 16)] = jnp.minimum(
                        jnp.maximum(v, 0), n_table - 1)
            copies = [
                pltpu.async_copy(
                    table_hbm.at[idx_v.at[pl.ds(j * ch, ch)]],
                    rows_v.at[pl.ds(j * ch, ch)], sems[j])
                for j in range(nch)
            ]
            for j in range(nch):
                copies[j].wait()
                pltpu.sync_copy(rows_v.at[pl.ds(j * ch, ch)],
                                out_hbm.at[pl.ds(base + j * ch, ch)])

        return k

    return (sc_meta,
            make_row_gather(PAD, T, True, "sc_gather_pad"),
            make_row_gather(T, PAD, False, "sc_gather_tok"))


def _group_body(te_ref, xs_ref, w1_ref, w2_ref, ws_ref, out_ref):
    h = _gelu_exact(jnp.dot(xs_ref[...].astype(jnp.bfloat16),
                            w1_ref[0].astype(jnp.bfloat16),
                            preferred_element_type=jnp.float32))
    y = jnp.dot(h.astype(jnp.bfloat16), w2_ref[0].astype(jnp.bfloat16),
                preferred_element_type=jnp.float32)
    out_ref[...] = y * ws_ref[...]


def _grouped_mlp(te, xs, w_fc1, w_fc2, ws_col):
    grid_spec = pltpu.PrefetchScalarGridSpec(
        num_scalar_prefetch=1,
        grid=(NT,),
        in_specs=[
            pl.BlockSpec((TT, H), lambda i, te_r: (i, 0)),
            pl.BlockSpec((1, H, F), lambda i, te_r: (te_r[i], 0, 0)),
            pl.BlockSpec((1, F, H), lambda i, te_r: (te_r[i], 0, 0)),
            pl.BlockSpec((TT, 1), lambda i, te_r: (i, 0)),
        ],
        out_specs=pl.BlockSpec((TT, H), lambda i, te_r: (i, 0)),
    )
    return pl.pallas_call(
        _group_body,
        grid_spec=grid_spec,
        out_shape=jax.ShapeDtypeStruct((PAD, H), jnp.float32),
    )(te, xs, w_fc1, w_fc2, ws_col)


def kernel(hidden_states, w_gate, w_fc1, w_fc2):
    x2d = hidden_states.reshape(T, H)
    sc_meta, sc_gather_pad, sc_gather_tok = _sc_kernels()
    logits, dest, w, te, xc = _router(x2d, w_gate)
    dest1 = dest.reshape(T)
    src, ws = sc_meta(dest1, w.reshape(T))
    xs = sc_gather_pad(xc, src)
    ys = _grouped_mlp(te.reshape(NT), xs, w_fc1, w_fc2, ws.reshape(PAD, 1))
    out2d = sc_gather_tok(ys, dest1)
    return out2d.reshape(B, S, H), logits


# speedup vs baseline: 1.5158x; 1.5158x over previous
import functools

import jax
import jax.numpy as jnp
import numpy as np
from jax import lax
from jax.experimental import pallas as pl
from jax.experimental.pallas import tpu as pltpu
from jax.experimental.pallas import tpu_sc as plsc

B = 1
S = 2048
T = 2048
H = 768
E = 16
F = 1024
TT = 128
NT = 32
PAD = NT * TT
NW = 32


def _gelu_exact(v):
    a = jnp.abs(v) * np.float32(0.7071067811865476)
    t = 1.0 / (1.0 + np.float32(0.3275911) * a)
    poly = t * (np.float32(0.254829592) + t * (np.float32(-0.284496736)
            + t * (np.float32(1.421413741) + t * (np.float32(-1.453152027)
            + t * np.float32(1.061405429)))))
    erf_a = 1.0 - poly * jnp.exp(-a * a)
    erf_v = jnp.where(v >= 0, erf_a, -erf_a)
    return np.float32(0.5) * v * (1.0 + erf_v)


def _router_body(x_ref, wg_ref, logits_ref, dest_ref, w_ref, te_ref):
    x = x_ref[...]
    wg = wg_ref[...]
    logits = jnp.dot(x, wg, preferred_element_type=jnp.float32)
    logits_ref[...] = logits
    m = jnp.max(logits, axis=1, keepdims=True)
    denom = jnp.sum(jnp.exp(logits - m), axis=1, keepdims=True)
    w_ref[...] = 1.0 / denom
    col = jax.lax.broadcasted_iota(jnp.int32, (T, E), 1)
    eid = jnp.min(jnp.where(logits == m, col, E), axis=1, keepdims=True)
    onehot = (col == eid).astype(jnp.float32)
    r = jax.lax.broadcasted_iota(jnp.int32, (TT, TT), 0)
    c = jax.lax.broadcasted_iota(jnp.int32, (TT, TT), 1)
    tril = (c <= r).astype(jnp.float32)
    blocks = []
    running = jnp.zeros((1, E), jnp.float32)
    for i in range(T // TT):
        csb = jnp.dot(tril, onehot[i * TT:(i + 1) * TT, :],
                      preferred_element_type=jnp.float32) + running
        running = csb[TT - 1:TT, :]
        blocks.append(csb)
    cs = jnp.concatenate(blocks, axis=0)
    counts = running
    padded = jnp.ceil(counts / TT) * TT
    er = jax.lax.broadcasted_iota(jnp.int32, (E, E), 0)
    ec = jax.lax.broadcasted_iota(jnp.int32, (E, E), 1)
    strict = (er < ec).astype(jnp.float32)
    base = jnp.dot(padded, strict, preferred_element_type=jnp.float32)
    pos = jnp.sum(onehot * cs, axis=1, keepdims=True) - 1.0
    tok_base = jnp.sum(onehot * base, axis=1, keepdims=True)
    dest_ref[...] = (tok_base + pos).astype(jnp.int32)
    ti = jax.lax.broadcasted_iota(jnp.int32, (NT, E), 0).astype(jnp.float32) * np.float32(TT)
    ecol = jax.lax.broadcasted_iota(jnp.int32, (NT, E), 1)
    active = jnp.logical_and(ti >= base, ti < base + padded)
    te_ref[...] = jnp.sum(jnp.where(active, ecol, 0), axis=1, keepdims=True)


def _router(x2d, w_gate):
    return pl.pallas_call(
        _router_body,
        out_shape=(
            jax.ShapeDtypeStruct((T, E), jnp.float32),
            jax.ShapeDtypeStruct((T, 1), jnp.int32),
            jax.ShapeDtypeStruct((T, 1), jnp.float32),
            jax.ShapeDtypeStruct((NT, 1), jnp.int32),
        ),
    )(x2d, w_gate)


@functools.cache
def _sc_kernels():
    mesh = plsc.VectorSubcoreMesh(core_axis_name="c", subcore_axis_name="s")

    @functools.partial(
        pl.kernel,
        out_type=(
            jax.ShapeDtypeStruct((PAD,), jnp.int32),
            jax.ShapeDtypeStruct((PAD,), jnp.float32),
        ),
        mesh=mesh,
        scratch_types=[
            pltpu.VMEM((T,), jnp.int32),
            pltpu.VMEM((T,), jnp.float32),
            pltpu.VMEM((PAD,), jnp.int32),
            pltpu.VMEM((PAD,), jnp.float32),
        ],
        compiler_params=pltpu.CompilerParams(needs_layout_passes=False),
        name="sc_meta",
    )
    def sc_meta(dest_hbm, w_hbm, src_hbm, ws_hbm, dest_v, w_v, src_v, ws_v):
        wid = lax.axis_index("s") * 2 + lax.axis_index("c")

        @pl.when(wid == 0)
        def _():
            pltpu.sync_copy(dest_hbm, dest_v)
            pltpu.sync_copy(w_hbm, w_v)
            lane = lax.broadcasted_iota(jnp.int32, (16,), 0)

            @plsc.parallel_loop(0, PAD // 16, unroll=8)
            def _init(i):
                src_v[pl.ds(i * 16, 16)] = lane + (i & (T // 16 - 1)) * 16

            @plsc.parallel_loop(0, T // 16, unroll=8)
            def _scat(i):
                idx = dest_v[pl.ds(i * 16, 16)]
                plsc.store_scatter(src_v, [idx], lane + i * 16)
                plsc.store_scatter(ws_v, [idx], w_v[pl.ds(i * 16, 16)])

            pltpu.sync_copy(src_v, src_hbm)
            pltpu.sync_copy(ws_v, ws_hbm)

    def make_row_gather(n_rows, n_table, clamp, name):
        rpt = n_rows // NW
        ch = 64
        nch = rpt // ch

        @functools.partial(
            pl.kernel,
            out_type=jax.ShapeDtypeStruct((n_rows, H), jnp.float32),
            mesh=mesh,
            scratch_types=[
                pltpu.VMEM((rpt,), jnp.int32),
                pltpu.VMEM((rpt, H), jnp.float32),
                [pltpu.SemaphoreType.DMA] * nch,
            ],
            name=name,
        )
        def k(table_hbm, idx_hbm, out_hbm, idx_v, rows_v, sems):
            wid = lax.axis_index("s") * 2 + lax.axis_index("c")
            base = wid * rpt
            pltpu.sync_copy(idx_hbm.at[pl.ds(base, rpt)], idx_v)
            if clamp:
                for j in range(rpt // 16):
                    v = idx_v[pl.ds(j * 16, 16)]
                    idx_v[pl.ds(j * 16, 16)] = jnp.minimum(
                        jnp.maximum(v, 0), n_table - 1)
            copies = [
                pltpu.async_copy(
                    table_hbm.at[idx_v.at[pl.ds(j * ch, ch)]],
                    rows_v.at[pl.ds(j * ch, ch)], sems[j])
                for j in range(nch)
            ]
            for j in range(nch):
                copies[j].wait()
                pltpu.sync_copy(rows_v.at[pl.ds(j * ch, ch)],
                                out_hbm.at[pl.ds(base + j * ch, ch)])

        return k

    return (sc_meta,
            make_row_gather(PAD, T, True, "sc_gather_pad"),
            make_row_gather(T, PAD, False, "sc_gather_tok"))


def _group_body(te_ref, xs_ref, w1_ref, w2_ref, ws_ref, out_ref):
    h = _gelu_exact(jnp.dot(xs_ref[...].astype(jnp.bfloat16),
                            w1_ref[0].astype(jnp.bfloat16),
                            preferred_element_type=jnp.float32))
    y = jnp.dot(h.astype(jnp.bfloat16), w2_ref[0].astype(jnp.bfloat16),
                preferred_element_type=jnp.float32)
    out_ref[...] = y * ws_ref[...]


def _grouped_mlp(te, xs, w_fc1, w_fc2, ws_col):
    grid_spec = pltpu.PrefetchScalarGridSpec(
        num_scalar_prefetch=1,
        grid=(NT,),
        in_specs=[
            pl.BlockSpec((TT, H), lambda i, te_r: (i, 0)),
            pl.BlockSpec((1, H, F), lambda i, te_r: (te_r[i], 0, 0)),
            pl.BlockSpec((1, F, H), lambda i, te_r: (te_r[i], 0, 0)),
            pl.BlockSpec((TT, 1), lambda i, te_r: (i, 0)),
        ],
        out_specs=pl.BlockSpec((TT, H), lambda i, te_r: (i, 0)),
    )
    return pl.pallas_call(
        _group_body,
        grid_spec=grid_spec,
        out_shape=jax.ShapeDtypeStruct((PAD, H), jnp.float32),
    )(te, xs, w_fc1, w_fc2, ws_col)


def kernel(hidden_states, w_gate, w_fc1, w_fc2):
    x2d = hidden_states.reshape(T, H)
    sc_meta, sc_gather_pad, sc_gather_tok = _sc_kernels()
    logits, dest, w, te = _router(x2d, w_gate)
    dest1 = dest.reshape(T)
    src, ws = sc_meta(dest1, w.reshape(T))
    xs = sc_gather_pad(x2d, src)
    ys = _grouped_mlp(te.reshape(NT), xs, w_fc1, w_fc2, ws.reshape(PAD, 1))
    out2d = sc_gather_tok(ys, dest1)
    return out2d.reshape(B, S, H), logits


# speedup vs baseline: 1.5525x; 1.0242x over previous
import functools

import jax
import jax.numpy as jnp
import numpy as np
from jax import lax
from jax.experimental import pallas as pl
from jax.experimental.pallas import tpu as pltpu
from jax.experimental.pallas import tpu_sc as plsc

B = 1
S = 2048
T = 2048
H = 768
E = 16
F = 1024
TT = 128
NT = 32
PAD = NT * TT
NW = 32


def _gelu_exact(v):
    a = jnp.abs(v) * np.float32(0.7071067811865476)
    t = 1.0 / (1.0 + np.float32(0.3275911) * a)
    poly = t * (np.float32(0.254829592) + t * (np.float32(-0.284496736)
            + t * (np.float32(1.421413741) + t * (np.float32(-1.453152027)
            + t * np.float32(1.061405429)))))
    erf_a = 1.0 - poly * jnp.exp(-a * a)
    erf_v = jnp.where(v >= 0, erf_a, -erf_a)
    return np.float32(0.5) * v * (1.0 + erf_v)


def _router_body(x_ref, wg_ref, logits_ref, dest_ref, w_ref, te_ref):
    x = x_ref[...]
    wg = wg_ref[...]
    logits = jnp.dot(x, wg, preferred_element_type=jnp.float32)
    logits_ref[...] = logits
    m = jnp.max(logits, axis=1, keepdims=True)
    denom = jnp.sum(jnp.exp(logits - m), axis=1, keepdims=True)
    w_ref[...] = 1.0 / denom
    col = jax.lax.broadcasted_iota(jnp.int32, (T, E), 1)
    eid = jnp.min(jnp.where(logits == m, col, E), axis=1, keepdims=True)
    onehot = (col == eid).astype(jnp.float32)
    r = jax.lax.broadcasted_iota(jnp.int32, (TT, TT), 0)
    c = jax.lax.broadcasted_iota(jnp.int32, (TT, TT), 1)
    tril = (c <= r).astype(jnp.float32)
    blocks = []
    running = jnp.zeros((1, E), jnp.float32)
    for i in range(T // TT):
        csb = jnp.dot(tril, onehot[i * TT:(i + 1) * TT, :],
                      preferred_element_type=jnp.float32) + running
        running = csb[TT - 1:TT, :]
        blocks.append(csb)
    cs = jnp.concatenate(blocks, axis=0)
    counts = running
    padded = jnp.ceil(counts / TT) * TT
    er = jax.lax.broadcasted_iota(jnp.int32, (E, E), 0)
    ec = jax.lax.broadcasted_iota(jnp.int32, (E, E), 1)
    strict = (er < ec).astype(jnp.float32)
    base = jnp.dot(padded, strict, preferred_element_type=jnp.float32)
    pos = jnp.sum(onehot * cs, axis=1, keepdims=True) - 1.0
    tok_base = jnp.sum(onehot * base, axis=1, keepdims=True)
    dest_ref[...] = (tok_base + pos).astype(jnp.int32)
    ti = jax.lax.broadcasted_iota(jnp.int32, (NT, E), 0).astype(jnp.float32) * np.float32(TT)
    ecol = jax.lax.broadcasted_iota(jnp.int32, (NT, E), 1)
    active = jnp.logical_and(ti >= base, ti < base + padded)
    emax = jnp.max(jnp.where(padded > 0, ecol[:1, :], 0), axis=1, keepdims=True)
    te = jnp.sum(jnp.where(active, ecol, 0), axis=1, keepdims=True)
    any_active = jnp.sum(jnp.where(active, 1, 0), axis=1, keepdims=True) > 0
    te_ref[...] = jnp.where(any_active, te, emax)


def _router(x2d, w_gate):
    return pl.pallas_call(
        _router_body,
        out_shape=(
            jax.ShapeDtypeStruct((T, E), jnp.float32),
            jax.ShapeDtypeStruct((T, 1), jnp.int32),
            jax.ShapeDtypeStruct((T, 1), jnp.float32),
            jax.ShapeDtypeStruct((NT, 1), jnp.int32),
        ),
    )(x2d, w_gate)


@functools.cache
def _sc_kernels():
    mesh = plsc.VectorSubcoreMesh(core_axis_name="c", subcore_axis_name="s")

    rpt = PAD // NW
    ch = 64
    nch = rpt // ch

    @functools.partial(
        pl.kernel,
        out_type=(
            jax.ShapeDtypeStruct((PAD, H), jnp.float32),
            jax.ShapeDtypeStruct((PAD,), jnp.float32),
        ),
        mesh=mesh,
        scratch_types=[
            pltpu.VMEM((T,), jnp.int32),
            pltpu.VMEM((T,), jnp.float32),
            pltpu.VMEM((PAD,), jnp.int32),
            pltpu.VMEM((PAD,), jnp.float32),
            pltpu.VMEM((rpt, H), jnp.float32),
            [pltpu.SemaphoreType.DMA] * nch,
        ],
        compiler_params=pltpu.CompilerParams(needs_layout_passes=False),
        name="sc_sort_gather",
    )
    def sc_sort_gather(x_hbm, dest_hbm, w_hbm, xs_hbm, ws_hbm,
                       dest_v, w_v, src_v, ws_v, rows_v, sems):
        wid = lax.axis_index("s") * 2 + lax.axis_index("c")
        base = wid * rpt
        lane = lax.broadcasted_iota(jnp.int32, (16,), 0)
        pltpu.sync_copy(dest_hbm, dest_v)

        @plsc.parallel_loop(0, PAD // 16, unroll=8)
        def _init(i):
            src_v[pl.ds(i * 16, 16)] = lane + (i & (T // 16 - 1)) * 16

        @plsc.parallel_loop(0, T // 16, unroll=8)
        def _scat(i):
            idx = dest_v[pl.ds(i * 16, 16)]
            plsc.store_scatter(src_v, [idx], lane + i * 16)

        copies = [
            pltpu.async_copy(
                x_hbm.at[src_v.at[pl.ds(base + j * ch, ch)]],
                rows_v.at[pl.ds(j * ch, ch)], sems[j])
            for j in range(nch)
        ]

        @pl.when(wid == 0)
        def _():
            pltpu.sync_copy(w_hbm, w_v)

            @plsc.parallel_loop(0, T // 16, unroll=8)
            def _scat_w(i):
                idx = dest_v[pl.ds(i * 16, 16)]
                plsc.store_scatter(ws_v, [idx], w_v[pl.ds(i * 16, 16)])

            pltpu.sync_copy(ws_v, ws_hbm)

        for j in range(nch):
            copies[j].wait()
            pltpu.sync_copy(rows_v.at[pl.ds(j * ch, ch)],
                            xs_hbm.at[pl.ds(base + j * ch, ch)])

    def make_row_gather(n_rows, n_table, clamp, name):
        rpt = n_rows // NW
        ch = 64
        nch = rpt // ch

        @functools.partial(
            pl.kernel,
            out_type=jax.ShapeDtypeStruct((n_rows, H), jnp.float32),
            mesh=mesh,
            scratch_types=[
                pltpu.VMEM((rpt,), jnp.int32),
                pltpu.VMEM((rpt, H), jnp.float32),
                [pltpu.SemaphoreType.DMA] * nch,
            ],
            name=name,
        )
        def k(table_hbm, idx_hbm, out_hbm, idx_v, rows_v, sems):
            wid = lax.axis_index("s") * 2 + lax.axis_index("c")
            base = wid * rpt
            pltpu.sync_copy(idx_hbm.at[pl.ds(base, rpt)], idx_v)
            if clamp:
                for j in range(rpt // 16):
                    v = idx_v[pl.ds(j * 16, 16)]
                    idx_v[pl.ds(j * 16, 16)] = jnp.minimum(
                        jnp.maximum(v, 0), n_table - 1)
            copies = [
                pltpu.async_copy(
                    table_hbm.at[idx_v.at[pl.ds(j * ch, ch)]],
                    rows_v.at[pl.ds(j * ch, ch)], sems[j])
                for j in range(nch)
            ]
            for j in range(nch):
                copies[j].wait()
                pltpu.sync_copy(rows_v.at[pl.ds(j * ch, ch)],
                                out_hbm.at[pl.ds(base + j * ch, ch)])

        return k

    return sc_sort_gather, make_row_gather(T, PAD, False, "sc_gather_tok")


def _group_body(te_ref, xs_ref, w1_ref, w2_ref, ws_ref, out_ref):
    h = _gelu_exact(jnp.dot(xs_ref[...].astype(jnp.bfloat16),
                            w1_ref[0].astype(jnp.bfloat16),
                            preferred_element_type=jnp.float32))
    y = jnp.dot(h.astype(jnp.bfloat16), w2_ref[0].astype(jnp.bfloat16),
                preferred_element_type=jnp.float32)
    out_ref[...] = y * ws_ref[...]


def _grouped_mlp(te, xs, w_fc1, w_fc2, ws_col):
    grid_spec = pltpu.PrefetchScalarGridSpec(
        num_scalar_prefetch=1,
        grid=(NT,),
        in_specs=[
            pl.BlockSpec((TT, H), lambda i, te_r: (i, 0)),
            pl.BlockSpec((1, H, F), lambda i, te_r: (te_r[i], 0, 0)),
            pl.BlockSpec((1, F, H), lambda i, te_r: (te_r[i], 0, 0)),
            pl.BlockSpec((TT, 1), lambda i, te_r: (i, 0)),
        ],
        out_specs=pl.BlockSpec((TT, H), lambda i, te_r: (i, 0)),
    )
    return pl.pallas_call(
        _group_body,
        grid_spec=grid_spec,
        out_shape=jax.ShapeDtypeStruct((PAD, H), jnp.float32),
    )(te, xs, w_fc1, w_fc2, ws_col)


def kernel(hidden_states, w_gate, w_fc1, w_fc2):
    x2d = hidden_states.reshape(T, H)
    sc_sort_gather, sc_gather_tok = _sc_kernels()
    logits, dest, w, te = _router(x2d, w_gate)
    dest1 = dest.reshape(T)
    xs, ws = sc_sort_gather(x2d, dest1, w.reshape(T))
    ys = _grouped_mlp(te.reshape(NT), xs, w_fc1, w_fc2, ws.reshape(PAD, 1))
    out2d = sc_gather_tok(ys, dest1)
    return out2d.reshape(B, S, H), logits


# speedup vs baseline: 1.6807x; 1.0825x over previous
import functools

import jax
import jax.numpy as jnp
import numpy as np
from jax import lax
from jax.experimental import pallas as pl
from jax.experimental.pallas import tpu as pltpu
from jax.experimental.pallas import tpu_sc as plsc

B = 1
S = 2048
T = 2048
H = 768
E = 16
F = 1024
TT = 128
NT = 32
PAD = NT * TT
NW = 32


def _gelu_exact(v):
    a = jnp.abs(v) * np.float32(0.7071067811865476)
    t = 1.0 / (1.0 + np.float32(0.3275911) * a)
    poly = t * (np.float32(0.254829592) + t * (np.float32(-0.284496736)
            + t * (np.float32(1.421413741) + t * (np.float32(-1.453152027)
            + t * np.float32(1.061405429)))))
    erf_a = 1.0 - poly * jnp.exp(-a * a)
    erf_v = jnp.where(v >= 0, erf_a, -erf_a)
    return np.float32(0.5) * v * (1.0 + erf_v)


def _router_body(x_ref, wg_ref, logits_ref, dest_ref, w_ref, te_ref):
    x = x_ref[...]
    wg = wg_ref[...]
    logits = jnp.dot(x, wg, preferred_element_type=jnp.float32)
    logits_ref[...] = logits
    m = jnp.max(logits, axis=1, keepdims=True)
    denom = jnp.sum(jnp.exp(logits - m), axis=1, keepdims=True)
    w_ref[...] = 1.0 / denom
    col = jax.lax.broadcasted_iota(jnp.int32, (T, E), 1)
    eid = jnp.min(jnp.where(logits == m, col, E), axis=1, keepdims=True)
    onehot = (col == eid).astype(jnp.float32)
    r = jax.lax.broadcasted_iota(jnp.int32, (TT, TT), 0)
    c = jax.lax.broadcasted_iota(jnp.int32, (TT, TT), 1)
    tril = (c <= r).astype(jnp.float32)
    blocks = []
    running = jnp.zeros((1, E), jnp.float32)
    for i in range(T // TT):
        csb = jnp.dot(tril, onehot[i * TT:(i + 1) * TT, :],
                      preferred_element_type=jnp.float32) + running
        running = csb[TT - 1:TT, :]
        blocks.append(csb)
    cs = jnp.concatenate(blocks, axis=0)
    counts = running
    padded = jnp.ceil(counts / TT) * TT
    er = jax.lax.broadcasted_iota(jnp.int32, (E, E), 0)
    ec = jax.lax.broadcasted_iota(jnp.int32, (E, E), 1)
    strict = (er < ec).astype(jnp.float32)
    base = jnp.dot(padded, strict, preferred_element_type=jnp.float32)
    pos = jnp.sum(onehot * cs, axis=1, keepdims=True) - 1.0
    tok_base = jnp.sum(onehot * base, axis=1, keepdims=True)
    dest_ref[...] = (tok_base + pos).astype(jnp.int32)
    ti = jax.lax.broadcasted_iota(jnp.int32, (NT, E), 0).astype(jnp.float32) * np.float32(TT)
    ecol = jax.lax.broadcasted_iota(jnp.int32, (NT, E), 1)
    active = jnp.logical_and(ti >= base, ti < base + padded)
    emax = jnp.max(jnp.where(padded > 0, ecol[:1, :], 0), axis=1, keepdims=True)
    te = jnp.sum(jnp.where(active, ecol, 0), axis=1, keepdims=True)
    any_active = jnp.sum(jnp.where(active, 1, 0), axis=1, keepdims=True) > 0
    na = (jnp.sum(padded, axis=1, keepdims=True) / np.float32(TT)).astype(jnp.int32)
    te_ref[...] = jnp.concatenate(
        [jnp.where(any_active, te, emax), na, na], axis=0)


def _router(x2d, w_gate):
    return pl.pallas_call(
        _router_body,
        out_shape=(
            jax.ShapeDtypeStruct((T, E), jnp.float32),
            jax.ShapeDtypeStruct((T, 1), jnp.int32),
            jax.ShapeDtypeStruct((T, 1), jnp.float32),
            jax.ShapeDtypeStruct((NT + 2, 1), jnp.int32),
        ),
    )(x2d, w_gate)


@functools.cache
def _sc_kernels():
    mesh = plsc.VectorSubcoreMesh(core_axis_name="c", subcore_axis_name="s")

    rpt = PAD // NW
    ch = 64
    nch = rpt // ch

    @functools.partial(
        pl.kernel,
        out_type=(
            jax.ShapeDtypeStruct((PAD, H), jnp.float32),
            jax.ShapeDtypeStruct((PAD,), jnp.float32),
        ),
        mesh=mesh,
        scratch_types=[
            pltpu.VMEM((T,), jnp.int32),
            pltpu.VMEM((T,), jnp.float32),
            pltpu.VMEM((PAD,), jnp.int32),
            pltpu.VMEM((PAD,), jnp.float32),
            pltpu.VMEM((rpt, H), jnp.float32),
            [pltpu.SemaphoreType.DMA] * nch,
        ],
        compiler_params=pltpu.CompilerParams(needs_layout_passes=False),
        name="sc_sort_gather",
    )
    def sc_sort_gather(x_hbm, dest_hbm, w_hbm, xs_hbm, ws_hbm,
                       dest_v, w_v, src_v, ws_v, rows_v, sems):
        wid = lax.axis_index("s") * 2 + lax.axis_index("c")
        base = wid * rpt
        lane = lax.broadcasted_iota(jnp.int32, (16,), 0)
        pltpu.sync_copy(dest_hbm, dest_v)

        @plsc.parallel_loop(0, PAD // 16, unroll=8)
        def _init(i):
            src_v[pl.ds(i * 16, 16)] = lane + (i & (T // 16 - 1)) * 16

        @plsc.parallel_loop(0, T // 16, unroll=8)
        def _scat(i):
            idx = dest_v[pl.ds(i * 16, 16)]
            plsc.store_scatter(src_v, [idx], lane + i * 16)

        copies = [
            pltpu.async_copy(
                x_hbm.at[src_v.at[pl.ds(base + j * ch, ch)]],
                rows_v.at[pl.ds(j * ch, ch)], sems[j])
            for j in range(nch)
        ]

        @pl.when(wid == 0)
        def _():
            pltpu.sync_copy(w_hbm, w_v)

            @plsc.parallel_loop(0, T // 16, unroll=8)
            def _scat_w(i):
                idx = dest_v[pl.ds(i * 16, 16)]
                plsc.store_scatter(ws_v, [idx], w_v[pl.ds(i * 16, 16)])

            pltpu.sync_copy(ws_v, ws_hbm)

        for j in range(nch):
            copies[j].wait()
            pltpu.sync_copy(rows_v.at[pl.ds(j * ch, ch)],
                            xs_hbm.at[pl.ds(base + j * ch, ch)])

    def make_row_gather(n_rows, n_table, clamp, name):
        rpt = n_rows // NW
        ch = 64
        nch = rpt // ch

        @functools.partial(
            pl.kernel,
            out_type=jax.ShapeDtypeStruct((n_rows, H), jnp.float32),
            mesh=mesh,
            scratch_types=[
                pltpu.VMEM((rpt,), jnp.int32),
                pltpu.VMEM((rpt, H), jnp.float32),
                [pltpu.SemaphoreType.DMA] * nch,
            ],
            name=name,
        )
        def k(table_hbm, idx_hbm, out_hbm, idx_v, rows_v, sems):
            wid = lax.axis_index("s") * 2 + lax.axis_index("c")
            base = wid * rpt
            pltpu.sync_copy(idx_hbm.at[pl.ds(base, rpt)], idx_v)
            if clamp:
                for j in range(rpt // 16):
                    v = idx_v[pl.ds(j * 16, 16)]
                    idx_v[pl.ds(j * 16, 16)] = jnp.minimum(
                        jnp.maximum(v, 0), n_table - 1)
            copies = [
                pltpu.async_copy(
                    table_hbm.at[idx_v.at[pl.ds(j * ch, ch)]],
                    rows_v.at[pl.ds(j * ch, ch)], sems[j])
                for j in range(nch)
            ]
            for j in range(nch):
                copies[j].wait()
                pltpu.sync_copy(rows_v.at[pl.ds(j * ch, ch)],
                                out_hbm.at[pl.ds(base + j * ch, ch)])

        return k

    return sc_sort_gather, make_row_gather(T, PAD, False, "sc_gather_tok")


def _group_body(te_ref, xs_ref, w1_ref, w2_ref, ws_ref, out_ref):
    @pl.when(pl.program_id(0) < te_ref[NT])
    def _():
        h = _gelu_exact(jnp.dot(xs_ref[...].astype(jnp.bfloat16),
                                w1_ref[0].astype(jnp.bfloat16),
                                preferred_element_type=jnp.float32))
        y = jnp.dot(h.astype(jnp.bfloat16), w2_ref[0].astype(jnp.bfloat16),
                    preferred_element_type=jnp.float32)
        out_ref[...] = y * ws_ref[...]


def _grouped_mlp(te, xs, w_fc1, w_fc2, ws_col):
    grid_spec = pltpu.PrefetchScalarGridSpec(
        num_scalar_prefetch=1,
        grid=(NT,),
        in_specs=[
            pl.BlockSpec((TT, H), lambda i, te_r: (jnp.minimum(i, te_r[NT] - 1), 0)),
            pl.BlockSpec((1, H, F), lambda i, te_r: (te_r[i], 0, 0)),
            pl.BlockSpec((1, F, H), lambda i, te_r: (te_r[i], 0, 0)),
            pl.BlockSpec((TT, 1), lambda i, te_r: (jnp.minimum(i, te_r[NT] - 1), 0)),
        ],
        out_specs=pl.BlockSpec(
            (TT, H), lambda i, te_r: (jnp.minimum(i, te_r[NT] - 1), 0)),
    )
    return pl.pallas_call(
        _group_body,
        grid_spec=grid_spec,
        out_shape=jax.ShapeDtypeStruct((PAD, H), jnp.float32),
    )(te, xs, w_fc1, w_fc2, ws_col)


def kernel(hidden_states, w_gate, w_fc1, w_fc2):
    x2d = hidden_states.reshape(T, H)
    sc_sort_gather, sc_gather_tok = _sc_kernels()
    logits, dest, w, te = _router(x2d, w_gate)
    dest1 = dest.reshape(T)
    xs, ws = sc_sort_gather(x2d, dest1, w.reshape(T))
    ys = _grouped_mlp(te.reshape(NT + 2), xs, w_fc1, w_fc2, ws.reshape(PAD, 1))
    out2d = sc_gather_tok(ys, dest1)
    return out2d.reshape(B, S, H), logits
